# baseline (speedup 1.0000x reference)
import functools

import numpy as np
import jax
import jax.numpy as jnp
from jax import lax
from jax.experimental import pallas as pl
from jax.experimental.pallas import tpu as pltpu
from jax.experimental.pallas import tpu_sc as plsc

_VOCAB = 1000000
_D = 64
_B = 1024
_S = 200
_NW = 32
_ROWS = _B * _S
_RPW = _ROWS // _NW
_CHUNK = 400
_NCHUNK = _RPW // _CHUNK
_REPS = _CHUNK // _S
_NGBUF = 3
_NOBUF = 2


def _pos_encoding() -> np.ndarray:
    pos = np.arange(_S, dtype=np.float32)[:, None]
    i = np.arange(_D // 2, dtype=np.float32)[None, :]
    angles = pos / np.power(10000.0, (2.0 * i) / _D)
    pe = np.zeros((_S, _D), dtype=np.float32)
    pe[:, 0::2] = np.sin(angles)
    pe[:, 1::2] = np.cos(angles)
    return pe


_MESH = plsc.VectorSubcoreMesh(core_axis_name="c", subcore_axis_name="s")


@functools.partial(
    pl.kernel,
    out_type=jax.ShapeDtypeStruct((_ROWS, _D), jnp.bfloat16),
    mesh=_MESH,
    compiler_params=pltpu.CompilerParams(use_tc_tiling_on_sc=False),
    scratch_types=[
        pltpu.VMEM((_RPW,), jnp.int32),
        pltpu.VMEM((_S, _D), jnp.bfloat16),
        pltpu.VMEM((_NGBUF, _CHUNK, _D), jnp.bfloat16),
        pltpu.VMEM((_NOBUF, _CHUNK, _D), jnp.bfloat16),
        pltpu.SemaphoreType.DMA,
        pltpu.SemaphoreType.DMA((_NGBUF,)),
        pltpu.SemaphoreType.DMA((_NOBUF,)),
    ],
)
def _emb_lookup(x_hbm, table_hbm, pe_hbm, out_hbm,
                idx_v, pe_v, gbuf, obuf, lsem, gsem, osem):
    wid = lax.axis_index("s") * 2 + lax.axis_index("c")
    base = wid * _RPW
    pltpu.async_copy(x_hbm.at[pl.ds(base, _RPW)], idx_v, lsem).wait()
    pltpu.async_copy(pe_hbm, pe_v, lsem).wait()

    def gather_desc(g):
        start = pl.multiple_of(g * _CHUNK, 8)
        return pltpu.make_async_copy(
            table_hbm.at[idx_v.at[pl.ds(start, _CHUNK)]],
            gbuf.at[g % _NGBUF], gsem.at[g % _NGBUF])

    def writeback_desc(g):
        start = pl.multiple_of(g * _CHUNK, 8)
        return pltpu.make_async_copy(
            obuf.at[g % _NOBUF], out_hbm.at[pl.ds(base + start, _CHUNK)],
            osem.at[g % _NOBUF])

    for g in range(_NGBUF):
        gather_desc(g).start()

    for g in range(_NCHUNK):
        gb = gbuf.at[g % _NGBUF]
        ob = obuf.at[g % _NOBUF]
        gather_desc(g).wait()
        if g >= _NOBUF:
            writeback_desc(g - _NOBUF).wait()

        def add_body(i, c):
            for rep in range(_REPS):
                row = rep * _S + i
                for k in range(2):
                    sl = pl.ds(k * 32, 32)
                    ob[row, sl] = gb[row, sl] + pe_v[i, sl]
            return c

        lax.fori_loop(0, _S, add_body, 0, unroll=2)

        writeback_desc(g).start()
        if g + _NGBUF < _NCHUNK:
            gather_desc(g + _NGBUF).start()

    for g in range(_NCHUNK - _NOBUF, _NCHUNK):
        writeback_desc(g).wait()


def kernel(x, table):
    pe = jnp.asarray(_pos_encoding().astype(jnp.bfloat16))
    out = _emb_lookup(x.reshape(-1), table.astype(jnp.bfloat16), pe)
    return out.astype(jnp.float32).reshape(_B, _S, _D)


# speedup vs baseline: 1.2266x; 1.2266x over previous
import functools

import numpy as np
import jax
import jax.numpy as jnp
from jax import lax
from jax.experimental import pallas as pl
from jax.experimental.pallas import tpu as pltpu
from jax.experimental.pallas import tpu_sc as plsc

_VOCAB = 1000000
_D = 64
_B = 1024
_S = 200
_NW = 32
_ROWS = _B * _S
_RPW = _ROWS // _NW
_CHUNK = 160
_NCHUNK = _RPW // _CHUNK
_NGBUF = 2
_NOBUF = 2
_L = 16


def _pos_encoding() -> np.ndarray:
    pos = np.arange(_S, dtype=np.float32)[:, None]
    i = np.arange(_D // 2, dtype=np.float32)[None, :]
    angles = pos / np.power(10000.0, (2.0 * i) / _D)
    pe = np.zeros((_S, _D), dtype=np.float32)
    pe[:, 0::2] = np.sin(angles)
    pe[:, 1::2] = np.cos(angles)
    return pe


_MESH = plsc.VectorSubcoreMesh(core_axis_name="c", subcore_axis_name="s")


@functools.partial(
    pl.kernel,
    out_type=jax.ShapeDtypeStruct((_ROWS, _D), jnp.float32),
    mesh=_MESH,
    compiler_params=pltpu.CompilerParams(use_tc_tiling_on_sc=True),
    scratch_types=[
        pltpu.VMEM((_RPW,), jnp.int32),
        pltpu.VMEM((_RPW + _L,), jnp.int32),
        pltpu.VMEM((_S, _D), jnp.float32),
        pltpu.VMEM((_NGBUF, _CHUNK, 128), jnp.float32),
        pltpu.VMEM((_NOBUF, _CHUNK, _D), jnp.float32),
        pltpu.SemaphoreType.DMA,
        pltpu.SemaphoreType.DMA((_NGBUF,)),
        pltpu.SemaphoreType.DMA((_NOBUF,)),
    ],
)
def _emb_lookup(x_hbm, tab_hbm, pe_hbm, out_hbm,
                slab_v, half_v, pe_v, gbuf, obuf, lsem, gsem, osem):
    wid = lax.axis_index("s") * 2 + lax.axis_index("c")
    base = wid * _RPW
    pltpu.async_copy(x_hbm.at[pl.ds(base, _RPW)], slab_v, lsem).wait()
    pltpu.async_copy(pe_hbm, pe_v, lsem).wait()

    def split_body(i, c):
        tok = slab_v[pl.ds(i * _L, _L)]
        slab_v[pl.ds(i * _L, _L)] = tok >> 1
        half_v[pl.ds(i * _L, _L)] = (tok & 1) * _D
        return c

    lax.fori_loop(0, _RPW // _L, split_body, 0, unroll=4)

    def gather_desc(g):
        start = pl.multiple_of(g * _CHUNK, 8)
        return pltpu.make_async_copy(
            tab_hbm.at[slab_v.at[pl.ds(start, _CHUNK)]],
            gbuf.at[g % _NGBUF], gsem.at[g % _NGBUF])

    def writeback_desc(g):
        start = pl.multiple_of(g * _CHUNK, 8)
        return pltpu.make_async_copy(
            obuf.at[g % _NOBUF], out_hbm.at[pl.ds(base + start, _CHUNK)],
            osem.at[g % _NOBUF])

    iota = lax.iota(jnp.int32, _L)
    for g in range(_NGBUF):
        gather_desc(g).start()

    for g in range(_NCHUNK):
        gb = gbuf.at[g % _NGBUF]
        ob = obuf.at[g % _NOBUF]
        gather_desc(g).wait()
        if g >= _NOBUF:
            writeback_desc(g - _NOBUF).wait()

        def row_body(i, c):
            off = half_v[pl.ds(g * _CHUNK + i, _L)][0]
            s = lax.rem(jnp.int32(g * _CHUNK) + i, jnp.int32(_S))
            for k in range(_D // _L):
                sl = pl.ds(k * _L, _L)
                v = gb[i, pl.ds(off + k * _L, _L)]
                ob[i, sl] = v + pe_v[s, sl]
            return c

        lax.fori_loop(0, _CHUNK, row_body, 0, unroll=2)

        writeback_desc(g).start()
        if g + _NGBUF < _NCHUNK:
            gather_desc(g + _NGBUF).start()

    for g in range(_NCHUNK - _NOBUF, _NCHUNK):
        writeback_desc(g).wait()


def kernel(x, table):
    pe = jnp.asarray(_pos_encoding())
    tab = table.reshape(_VOCAB // 2, 128)
    out = _emb_lookup(x.reshape(-1), tab, pe)
    return out.reshape(_B, _S, _D)


# speedup vs baseline: 1.2547x; 1.0229x over previous
import functools

import numpy as np
import jax
import jax.numpy as jnp
from jax import lax
from jax.experimental import pallas as pl
from jax.experimental.pallas import tpu as pltpu
from jax.experimental.pallas import tpu_sc as plsc

_VOCAB = 1000000
_D = 64
_B = 1024
_S = 200
_NW = 32
_ROWS = _B * _S
_RPW = _ROWS // _NW
_CHUNK = 200
_NCHUNK = _RPW // _CHUNK
_NBUF = 2
_L = 16


def _pos_encoding() -> np.ndarray:
    pos = np.arange(_S, dtype=np.float32)[:, None]
    i = np.arange(_D // 2, dtype=np.float32)[None, :]
    angles = pos / np.power(10000.0, (2.0 * i) / _D)
    pe = np.zeros((_S, _D), dtype=np.float32)
    pe[:, 0::2] = np.sin(angles)
    pe[:, 1::2] = np.cos(angles)
    return pe


_MESH = plsc.VectorSubcoreMesh(core_axis_name="c", subcore_axis_name="s")


@functools.partial(
    pl.kernel,
    out_type=jax.ShapeDtypeStruct((_ROWS, _D), jnp.float32),
    mesh=_MESH,
    compiler_params=pltpu.CompilerParams(use_tc_tiling_on_sc=True),
    scratch_types=[
        pltpu.VMEM((_RPW,), jnp.int32),
        pltpu.VMEM((_S * _D,), jnp.float32),
        pltpu.VMEM((_NBUF, _CHUNK, 128), jnp.float32),
        pltpu.VMEM((_NBUF, _CHUNK, _D), jnp.float32),
        pltpu.SemaphoreType.DMA,
        pltpu.SemaphoreType.DMA((_NBUF,)),
        pltpu.SemaphoreType.DMA((_NBUF,)),
    ],
)
def _emb_lookup(x_hbm, tab_hbm, pe_hbm, out_hbm,
                idx_v, pe_v, bufs, obufs, lsem, gsem, osem):
    wid = lax.axis_index("s") * 2 + lax.axis_index("c")
    base = wid * _RPW
    pltpu.async_copy(x_hbm.at[pl.ds(base, _RPW)], idx_v, lsem).wait()
    pltpu.async_copy(pe_hbm, pe_v, lsem).wait()

    def gather_desc(g):
        start = pl.multiple_of(g * _CHUNK, 8)
        return pltpu.make_async_copy(
            tab_hbm.at[idx_v.at[pl.ds(start, _CHUNK)]],
            bufs.at[g % _NBUF], gsem.at[g % _NBUF])

    def writeback_desc(g):
        start = pl.multiple_of(g * _CHUNK, 8)
        return pltpu.make_async_copy(
            obufs.at[g % _NBUF],
            out_hbm.at[pl.ds(base + start, _CHUNK)],
            osem.at[g % _NBUF])

    gather_desc(0).start()
    gather_desc(1).start()

    for g in range(_NCHUNK):
        gb = bufs.at[g % _NBUF]
        ob = obufs.at[g % _NBUF]
        gather_desc(g).wait()
        if g >= 2:
            writeback_desc(g - 2).wait()

        def row_body(i, c):
            pe_off = pl.multiple_of(i * _D, 8)
            for k in range(_D // _L):
                sl = pl.ds(k * _L, _L)
                ob[i, sl] = gb[i, sl] + pe_v[pl.ds(pe_off + k * _L, _L)]
            return c

        lax.fori_loop(0, _CHUNK, row_body, 0, unroll=2)

        writeback_desc(g).start()
        if g + 2 < _NCHUNK:
            gather_desc(g + 2).start()

    for g in range(_NCHUNK - 2, _NCHUNK):
        writeback_desc(g).wait()


def kernel(x, table):
    pe = jnp.asarray(_pos_encoding().reshape(-1))
    tab = jnp.concatenate([table, table], axis=1)
    out = _emb_lookup(x.reshape(-1), tab, pe)
    return out.reshape(_B, _S, _D)


# speedup vs baseline: 1.3621x; 1.0856x over previous
import functools

import numpy as np
import jax
import jax.numpy as jnp
from jax import lax
from jax.experimental import pallas as pl
from jax.experimental.pallas import tpu as pltpu
from jax.experimental.pallas import tpu_sc as plsc

_VOCAB = 1000000
_D = 64
_B = 1024
_S = 200
_NW = 32
_ROWS = _B * _S
_RPW = _ROWS // _NW
_CHUNK = 400
_NCHUNK = _RPW // _CHUNK
_REPS = _CHUNK // _S
_LANES = 16


def _pos_encoding() -> jnp.ndarray:
    pos = np.arange(_S, dtype=np.float32)[:, None]
    i = np.arange(_D // 2, dtype=np.float32)[None, :]
    angles = pos / np.power(10000.0, (2.0 * i) / _D)
    pe = np.zeros((_S, _D), dtype=np.float32)
    pe[:, 0::2] = np.sin(angles)
    pe[:, 1::2] = np.cos(angles)
    return jnp.asarray(pe)


_MESH = plsc.VectorSubcoreMesh(core_axis_name="c", subcore_axis_name="s")


@functools.partial(
    pl.kernel,
    out_type=jax.ShapeDtypeStruct((_ROWS, _D), jnp.float32),
    mesh=_MESH,
    compiler_params=pltpu.CompilerParams(use_tc_tiling_on_sc=False),
    scratch_types=[
        pltpu.VMEM((_RPW,), jnp.int32),
        pltpu.VMEM((_S, _D), jnp.float32),
        pltpu.VMEM((_CHUNK, _D), jnp.float32),
        pltpu.SemaphoreType.DMA,
    ],
)
def _emb_lookup(x_hbm, table_hbm, pe_hbm, out_hbm, idx_v, pe_v, buf_v, gsem):
    wid = lax.axis_index("s") * 2 + lax.axis_index("c")
    base = wid * _RPW
    pltpu.sync_copy(x_hbm.at[pl.ds(base, _RPW)], idx_v)
    pltpu.sync_copy(pe_hbm, pe_v)

    def chunk_body(g, carry):
        start = pl.multiple_of(g * _CHUNK, 8)
        pltpu.async_copy(
            table_hbm.at[idx_v.at[pl.ds(start, _CHUNK)]], buf_v, gsem
        ).wait()

        def add_body(i, c):
            for rep in range(_REPS):
                row = rep * _S + i
                for k in range(_D // _LANES):
                    sl = pl.ds(k * _LANES, _LANES)
                    plsc.addupdate(buf_v.at[row, sl], pe_v[i, sl])
            return c

        lax.fori_loop(0, _S, add_body, 0, unroll=2)

        pltpu.sync_copy(buf_v, out_hbm.at[pl.ds(base + start, _CHUNK)])
        return carry

    lax.fori_loop(0, _NCHUNK, chunk_body, 0)


def kernel(x, table):
    pe = _pos_encoding()
    out = _emb_lookup(x.reshape(-1), table, pe)
    return out.reshape(_B, _S, _D)
